# baseline (speedup 1.0000x reference)
import functools

import jax
import jax.numpy as jnp
from jax import lax
from jax.experimental import pallas as pl
from jax.experimental.pallas import tpu as pltpu
from jax.experimental.pallas import tpu_sc as plsc

_BATCH = 16
_MAX_CTX = 4096
_CHUNK_LEN = 1024
_LANES = 16
_ALIGN = 8


def _body(tokens_hbm, pool_hbm, starts_hbm, out_hbm,
          pool_v, starts_v, buf_v, out_v, sem_a, sem_b):
    i = lax.axis_index("s")

    cp_a = pltpu.async_copy(pool_hbm, pool_v, sem_a)
    cp_b = pltpu.async_copy(starts_hbm, starts_v, sem_b)
    cp_a.wait()
    cp_b.wait()

    lanes = lax.iota(jnp.int32, _LANES)
    sel = lanes == i
    pool_i = jnp.sum(jnp.where(sel, pool_v[...], 0))
    start_i = jnp.sum(jnp.where(sel, starts_v[...], 0))

    base = pool_i * _MAX_CTX + start_i
    al = (base // _ALIGN) * _ALIGN
    r = base - al

    pltpu.sync_copy(tokens_hbm.at[pl.ds(al, _CHUNK_LEN + _ALIGN)],
                    buf_v.at[pl.ds(0, _CHUNK_LEN + _ALIGN)])

    @pl.loop(0, _CHUNK_LEN // _LANES, unroll=8)
    def _realign(j):
        vals = plsc.load_gather(buf_v, [r + j * _LANES + lanes])
        out_v[pl.ds(pl.multiple_of(j * _LANES, _LANES), _LANES)] = vals

    out_off = pl.multiple_of(i * _CHUNK_LEN, _CHUNK_LEN)
    pltpu.sync_copy(out_v, out_hbm.at[pl.ds(out_off, _CHUNK_LEN)])


_gather = functools.partial(
    pl.kernel,
    mesh=plsc.VectorSubcoreMesh(core_axis_name="c", subcore_axis_name="s",
                                num_cores=1),
    out_type=jax.ShapeDtypeStruct((_BATCH * _CHUNK_LEN,), jnp.int32),
    scratch_types=[
        pltpu.VMEM((_LANES,), jnp.int32),
        pltpu.VMEM((_LANES,), jnp.int32),
        pltpu.VMEM((_CHUNK_LEN + _LANES,), jnp.int32),
        pltpu.VMEM((_CHUNK_LEN,), jnp.int32),
        pltpu.SemaphoreType.DMA,
        pltpu.SemaphoreType.DMA,
    ],
    compiler_params=pltpu.CompilerParams(needs_layout_passes=False),
)(_body)


def kernel(req_to_token, req_pool_indices, chunk_starts, chunk_seq_lens,
           chunk_cu_seq_lens):
    del chunk_seq_lens, chunk_cu_seq_lens
    flat = req_to_token.reshape(-1)
    return _gather(flat, req_pool_indices, chunk_starts)


# speedup vs baseline: 1.0026x; 1.0026x over previous
import functools

import jax
import jax.numpy as jnp
from jax import lax
from jax.experimental import pallas as pl
from jax.experimental.pallas import tpu as pltpu
from jax.experimental.pallas import tpu_sc as plsc

_BATCH = 16
_MAX_CTX = 4096
_CHUNK_LEN = 1024
_LANES = 16
_ALIGN = 8


def _body(tokens_hbm, pool_hbm, starts_hbm, out_hbm,
          pool_v, starts_v, buf_v, out_v, sem_a, sem_b):
    i = lax.axis_index("s")

    cp_a = pltpu.async_copy(pool_hbm, pool_v, sem_a)
    cp_b = pltpu.async_copy(starts_hbm, starts_v, sem_b)
    cp_a.wait()
    cp_b.wait()

    lanes = lax.iota(jnp.int32, _LANES)
    sel = lanes == i
    pool_i = jnp.sum(jnp.where(sel, pool_v[...], 0))
    start_i = jnp.sum(jnp.where(sel, starts_v[...], 0))

    base = pool_i * _MAX_CTX + start_i
    al = (base // _ALIGN) * _ALIGN
    r = base - al

    pltpu.sync_copy(tokens_hbm.at[pl.ds(al, _CHUNK_LEN + _ALIGN)],
                    buf_v.at[pl.ds(0, _CHUNK_LEN + _ALIGN)])

    @pl.loop(0, _CHUNK_LEN // _LANES)
    def _realign(j):
        vals = plsc.load_gather(buf_v, [r + j * _LANES + lanes])
        out_v[pl.ds(pl.multiple_of(j * _LANES, _LANES), _LANES)] = vals

    out_off = pl.multiple_of(i * _CHUNK_LEN, _CHUNK_LEN)
    pltpu.sync_copy(out_v, out_hbm.at[pl.ds(out_off, _CHUNK_LEN)])


_gather = functools.partial(
    pl.kernel,
    mesh=plsc.VectorSubcoreMesh(core_axis_name="c", subcore_axis_name="s",
                                num_cores=1),
    out_type=jax.ShapeDtypeStruct((_BATCH * _CHUNK_LEN,), jnp.int32),
    scratch_types=[
        pltpu.VMEM((_LANES,), jnp.int32),
        pltpu.VMEM((_LANES,), jnp.int32),
        pltpu.VMEM((_CHUNK_LEN + _LANES,), jnp.int32),
        pltpu.VMEM((_CHUNK_LEN,), jnp.int32),
        pltpu.SemaphoreType.DMA,
        pltpu.SemaphoreType.DMA,
    ],
    compiler_params=pltpu.CompilerParams(needs_layout_passes=False),
)(_body)


def kernel(req_to_token, req_pool_indices, chunk_starts, chunk_seq_lens,
           chunk_cu_seq_lens):
    del chunk_seq_lens, chunk_cu_seq_lens
    flat = req_to_token.reshape(-1)
    return _gather(flat, req_pool_indices, chunk_starts)


# speedup vs baseline: 1.0078x; 1.0051x over previous
import functools

import jax
import jax.numpy as jnp
from jax import lax
from jax.experimental import pallas as pl
from jax.experimental.pallas import tpu as pltpu
from jax.experimental.pallas import tpu_sc as plsc

_BATCH = 16
_MAX_CTX = 4096
_CHUNK_LEN = 1024
_LANES = 16
_ALIGN = 8


def _body(tokens_hbm, pool_hbm, starts_hbm, out_hbm,
          pool_v, starts_v, buf_v, out_v, sem_a, sem_b):
    i = lax.axis_index("s")

    cp_a = pltpu.async_copy(pool_hbm, pool_v, sem_a)
    cp_b = pltpu.async_copy(starts_hbm, starts_v, sem_b)
    cp_a.wait()
    cp_b.wait()

    lanes = lax.iota(jnp.int32, _LANES)
    sel = lanes == i
    pool_i = jnp.sum(jnp.where(sel, pool_v[...], 0))
    start_i = jnp.sum(jnp.where(sel, starts_v[...], 0))

    base = pool_i * _MAX_CTX + start_i
    al = (base // _ALIGN) * _ALIGN
    r = base - al

    pltpu.sync_copy(tokens_hbm.at[pl.ds(al, _CHUNK_LEN + _ALIGN)],
                    buf_v.at[pl.ds(0, _CHUNK_LEN + _ALIGN)])

    half = _CHUNK_LEN // 2
    out_off = pl.multiple_of(i * _CHUNK_LEN, _CHUNK_LEN)

    @pl.loop(0, half // _LANES)
    def _realign_a(j):
        vals = plsc.load_gather(buf_v, [r + j * _LANES + lanes])
        out_v[pl.ds(pl.multiple_of(j * _LANES, _LANES), _LANES)] = vals

    cp_oa = pltpu.async_copy(out_v.at[pl.ds(0, half)],
                             out_hbm.at[pl.ds(out_off, half)], sem_a)

    @pl.loop(half // _LANES, _CHUNK_LEN // _LANES)
    def _realign_b(j):
        vals = plsc.load_gather(buf_v, [r + j * _LANES + lanes])
        out_v[pl.ds(pl.multiple_of(j * _LANES, _LANES), _LANES)] = vals

    cp_ob = pltpu.async_copy(out_v.at[pl.ds(half, half)],
                             out_hbm.at[pl.ds(out_off + half, half)], sem_b)
    cp_oa.wait()
    cp_ob.wait()


_gather = functools.partial(
    pl.kernel,
    mesh=plsc.VectorSubcoreMesh(core_axis_name="c", subcore_axis_name="s",
                                num_cores=1),
    out_type=jax.ShapeDtypeStruct((_BATCH * _CHUNK_LEN,), jnp.int32),
    scratch_types=[
        pltpu.VMEM((_LANES,), jnp.int32),
        pltpu.VMEM((_LANES,), jnp.int32),
        pltpu.VMEM((_CHUNK_LEN + _LANES,), jnp.int32),
        pltpu.VMEM((_CHUNK_LEN,), jnp.int32),
        pltpu.SemaphoreType.DMA,
        pltpu.SemaphoreType.DMA,
    ],
    compiler_params=pltpu.CompilerParams(needs_layout_passes=False),
)(_body)


def kernel(req_to_token, req_pool_indices, chunk_starts, chunk_seq_lens,
           chunk_cu_seq_lens):
    del chunk_seq_lens, chunk_cu_seq_lens
    flat = req_to_token.reshape(-1)
    return _gather(flat, req_pool_indices, chunk_starts)
